# baseline (speedup 1.0000x reference)
import functools

import jax
import jax.numpy as jnp
from jax import lax
from jax.experimental import pallas as pl
from jax.experimental.pallas import tpu as pltpu
from jax.experimental.pallas import tpu_sc as plsc

EPS = 1e-5

_BN = 400
_CHUNKS = 5


def _sc_gather(table, idx_flat):
    n_rows, d = table.shape
    b = idx_flat.shape[0]
    nw = 32
    bpw = b // nw
    ch = 200
    nit = bpw // ch
    mesh = plsc.VectorSubcoreMesh(core_axis_name="c", subcore_axis_name="s")

    @functools.partial(
        pl.kernel,
        mesh=mesh,
        compiler_params=pltpu.CompilerParams(use_tc_tiling_on_sc=False),
        out_type=jax.ShapeDtypeStruct((b, d), table.dtype),
        scratch_types=[
            pltpu.VMEM((ch,), jnp.int32),
            pltpu.VMEM((ch,), jnp.int32),
            pltpu.VMEM((ch, d), table.dtype),
            pltpu.VMEM((ch, d), table.dtype),
            pltpu.SemaphoreType.DMA,
            pltpu.SemaphoreType.DMA,
            pltpu.SemaphoreType.DMA,
            pltpu.SemaphoreType.DMA,
        ],
    )
    def gather_kernel(table_hbm, idx_hbm, out_hbm, idx_v0, idx_v1,
                      rows_v0, rows_v1, sem_g0, sem_g1, sem_o0, sem_o1):
        wid = lax.axis_index("s") * 2 + lax.axis_index("c")
        base = wid * bpw

        def body(j, carry):
            off0 = base + (2 * j) * ch
            off1 = off0 + ch
            pltpu.sync_copy(idx_hbm.at[pl.ds(off0, ch)], idx_v0)
            g0 = pltpu.async_copy(table_hbm.at[idx_v0], rows_v0, sem_g0)
            pltpu.sync_copy(idx_hbm.at[pl.ds(off1, ch)], idx_v1)
            g0.wait()
            o0 = pltpu.async_copy(rows_v0, out_hbm.at[pl.ds(off0, ch)], sem_o0)
            g1 = pltpu.async_copy(table_hbm.at[idx_v1], rows_v1, sem_g1)
            g1.wait()
            o1 = pltpu.async_copy(rows_v1, out_hbm.at[pl.ds(off1, ch)], sem_o1)
            o0.wait()
            o1.wait()
            return carry

        lax.fori_loop(0, nit // 2, body, 0)

        @pl.when((nit % 2) == 1)
        def _tail():
            off = base + (nit - 1) * ch
            pltpu.sync_copy(idx_hbm.at[pl.ds(off, ch)], idx_v0)
            pltpu.async_copy(table_hbm.at[idx_v0], rows_v0, sem_g0).wait()
            pltpu.sync_copy(rows_v0, out_hbm.at[pl.ds(off, ch)])

    return gather_kernel(table, idx_flat)


def _edge_y(atom_ref, g_ref, nbr_ref, wa_ref, wg_ref, wf_ref, b_ref, bn, m):
    ya = jnp.dot(atom_ref[...], wa_ref[...], preferred_element_type=jnp.float32)
    yg = jnp.dot(g_ref[...], wg_ref[...], preferred_element_type=jnp.float32)
    yf = jnp.dot(nbr_ref[...], wf_ref[...], preferred_element_type=jnp.float32)
    y = (yg + yf + b_ref[...]).reshape(bn, m, 128)
    return y + ya[:, None, :]


def _pass12(atom, g2, nbr2, wa, wg, wf, bvec, gamma1, beta1, n, m):
    bn = _BN
    bm = bn * m
    grid = (2, n // bn)
    cnt = float(n * m)

    def body(atom_ref, g_ref, nbr_ref, wa_ref, wg_ref, wf_ref, b_ref,
             gam_ref, bet_ref, s_ref, t1_ref, t2_ref, s1_acc, s2_acc,
             sc_acc, sh_acc):
        p = pl.program_id(0)
        i = pl.program_id(1)
        y = _edge_y(atom_ref, g_ref, nbr_ref, wa_ref, wg_ref, wf_ref, b_ref,
                    bn, m)

        @pl.when(p == 0)
        def _():
            s1 = jnp.sum(y, axis=(0, 1))[None, :]
            s2 = jnp.sum(y * y, axis=(0, 1))[None, :]

            @pl.when(i == 0)
            def _():
                s1_acc[...] = jnp.zeros_like(s1_acc)
                s2_acc[...] = jnp.zeros_like(s2_acc)

            s1_acc[...] += s1
            s2_acc[...] += s2

        @pl.when(p == 1)
        def _():
            @pl.when(i == 0)
            def _():
                mean1 = s1_acc[...] / cnt
                var1 = s2_acc[...] / cnt - mean1 * mean1
                scale = gam_ref[...] * lax.rsqrt(var1 + EPS)
                sc_acc[...] = scale
                sh_acc[...] = bet_ref[...] - mean1 * scale
                t1_ref[...] = jnp.zeros_like(t1_ref)
                t2_ref[...] = jnp.zeros_like(t2_ref)

            yb = y * sc_acc[...].reshape(1, 1, 128) + sh_acc[...].reshape(1, 1, 128)
            log2e = 1.4426950408889634
            ln2 = 0.6931471805599453
            filt = 1.0 / (1.0 + jnp.exp2(yb[..., :64] * (-log2e)))
            core = jnp.log2(1.0 + jnp.exp2(yb[..., 64:] * log2e)) * ln2
            sb = jnp.sum(filt * core, axis=1)
            s_ref[...] = sb
            t1_ref[...] += jnp.sum(sb, axis=0)[None, :]
            t2_ref[...] += jnp.sum(sb * sb, axis=0)[None, :]

    return pl.pallas_call(
        body,
        grid=grid,
        in_specs=[
            pl.BlockSpec((bn, 64), lambda p, i: (i, 0)),
            pl.BlockSpec((bm, 64), lambda p, i: (i, 0)),
            pl.BlockSpec((bm, 41), lambda p, i: (i, 0)),
            pl.BlockSpec((64, 128), lambda p, i: (0, 0)),
            pl.BlockSpec((64, 128), lambda p, i: (0, 0)),
            pl.BlockSpec((41, 128), lambda p, i: (0, 0)),
            pl.BlockSpec((1, 128), lambda p, i: (0, 0)),
            pl.BlockSpec((1, 128), lambda p, i: (0, 0)),
            pl.BlockSpec((1, 128), lambda p, i: (0, 0)),
        ],
        out_specs=[
            pl.BlockSpec((bn, 64), lambda p, i: (i, 0)),
            pl.BlockSpec((1, 64), lambda p, i: (0, 0)),
            pl.BlockSpec((1, 64), lambda p, i: (0, 0)),
        ],
        out_shape=[
            jax.ShapeDtypeStruct((n, 64), jnp.float32),
            jax.ShapeDtypeStruct((1, 64), jnp.float32),
            jax.ShapeDtypeStruct((1, 64), jnp.float32),
        ],
        scratch_shapes=[
            pltpu.VMEM((1, 128), jnp.float32),
            pltpu.VMEM((1, 128), jnp.float32),
            pltpu.VMEM((1, 128), jnp.float32),
            pltpu.VMEM((1, 128), jnp.float32),
        ],
    )(atom, g2, nbr2, wa, wg, wf, bvec, gamma1, beta1)


def _softplus(x):
    return jnp.maximum(x, 0.0) + jnp.log1p(jnp.exp(-jnp.abs(x)))


def _pass3(atom, s, scale2, shift2, n):
    bn = 5000
    grid = (n // bn,)

    def body(atom_ref, s_ref, sc_ref, sh_ref, out_ref):
        z = atom_ref[...] + s_ref[...] * sc_ref[...] + sh_ref[...]
        out_ref[...] = _softplus(z)

    return pl.pallas_call(
        body,
        grid=grid,
        in_specs=[
            pl.BlockSpec((bn, 64), lambda i: (i, 0)),
            pl.BlockSpec((bn, 64), lambda i: (i, 0)),
            pl.BlockSpec((1, 64), lambda i: (0, 0)),
            pl.BlockSpec((1, 64), lambda i: (0, 0)),
        ],
        out_specs=pl.BlockSpec((bn, 64), lambda i: (i, 0)),
        out_shape=jax.ShapeDtypeStruct((n, 64), jnp.float32),
    )(atom, s, scale2, shift2)


def kernel(atom_fea, nbr_fea, nbr_fea_idx, W_full, b_full,
           bn1_gamma, bn1_beta, bn2_gamma, bn2_beta):
    n, m = nbr_fea_idx.shape
    idx_flat = nbr_fea_idx.reshape(-1).astype(jnp.int32)
    nbr2 = nbr_fea.reshape(n * m, 41).astype(jnp.bfloat16)

    wa = W_full[:, :64].T
    wg = W_full[:, 64:128].T
    wf = W_full[:, 128:].T.astype(jnp.bfloat16)
    bvec = b_full.reshape(1, 128)

    g2 = _sc_gather(atom_fea, idx_flat)

    s, t1, t2 = _pass12(atom_fea, g2, nbr2, wa, wg, wf, bvec,
                        bn1_gamma.reshape(1, 128), bn1_beta.reshape(1, 128),
                        n, m)
    mean2 = t1 / float(n)
    var2 = t2 / float(n) - mean2 * mean2
    scale2 = bn2_gamma[None, :] * lax.rsqrt(var2 + EPS)
    shift2 = bn2_beta[None, :] - mean2 * scale2

    return _pass3(atom_fea, s, scale2, shift2, n)


# speedup vs baseline: 1.0872x; 1.0872x over previous
import functools

import jax
import jax.numpy as jnp
from jax import lax
from jax.experimental import pallas as pl
from jax.experimental.pallas import tpu as pltpu
from jax.experimental.pallas import tpu_sc as plsc

EPS = 1e-5

_BN = 400
_CHUNKS = 5


def _sc_gather(table, idx_flat):
    n_rows, d = table.shape
    b = idx_flat.shape[0]
    nw = 32
    bpw = b // nw
    ch = 200
    nit = bpw // ch
    mesh = plsc.VectorSubcoreMesh(core_axis_name="c", subcore_axis_name="s")

    @functools.partial(
        pl.kernel,
        mesh=mesh,
        compiler_params=pltpu.CompilerParams(use_tc_tiling_on_sc=False),
        out_type=jax.ShapeDtypeStruct((b, d), table.dtype),
        scratch_types=[
            pltpu.VMEM((ch,), jnp.int32),
            pltpu.VMEM((ch,), jnp.int32),
            pltpu.VMEM((ch, d), table.dtype),
            pltpu.VMEM((ch, d), table.dtype),
            pltpu.SemaphoreType.DMA,
            pltpu.SemaphoreType.DMA,
            pltpu.SemaphoreType.DMA,
            pltpu.SemaphoreType.DMA,
        ],
    )
    def gather_kernel(table_hbm, idx_hbm, out_hbm, idx_v0, idx_v1,
                      rows_v0, rows_v1, sem_g0, sem_g1, sem_o0, sem_o1):
        wid = lax.axis_index("s") * 2 + lax.axis_index("c")
        base = wid * bpw

        def body(j, carry):
            off0 = base + (2 * j) * ch
            off1 = off0 + ch
            pltpu.sync_copy(idx_hbm.at[pl.ds(off0, ch)], idx_v0)
            g0 = pltpu.async_copy(table_hbm.at[idx_v0], rows_v0, sem_g0)
            pltpu.sync_copy(idx_hbm.at[pl.ds(off1, ch)], idx_v1)
            g0.wait()
            o0 = pltpu.async_copy(rows_v0, out_hbm.at[pl.ds(off0, ch)], sem_o0)
            g1 = pltpu.async_copy(table_hbm.at[idx_v1], rows_v1, sem_g1)
            g1.wait()
            o1 = pltpu.async_copy(rows_v1, out_hbm.at[pl.ds(off1, ch)], sem_o1)
            o0.wait()
            o1.wait()
            return carry

        lax.fori_loop(0, nit // 2, body, 0)

        @pl.when((nit % 2) == 1)
        def _tail():
            off = base + (nit - 1) * ch
            pltpu.sync_copy(idx_hbm.at[pl.ds(off, ch)], idx_v0)
            pltpu.async_copy(table_hbm.at[idx_v0], rows_v0, sem_g0).wait()
            pltpu.sync_copy(rows_v0, out_hbm.at[pl.ds(off, ch)])

    return gather_kernel(table, idx_flat)


def _edge_y(atom_ref, g_ref, nbr_ref, wa_ref, wg_ref, wf_ref, b_ref, bn, m):
    ya = jnp.dot(atom_ref[...], wa_ref[...], preferred_element_type=jnp.float32)
    yg = jnp.dot(g_ref[...], wg_ref[...], preferred_element_type=jnp.float32)
    yf = jnp.dot(nbr_ref[...], wf_ref[...], preferred_element_type=jnp.float32)
    y = (yg + yf + b_ref[...]).reshape(bn, m, 128)
    return y + ya[:, None, :]


def _pass1(atom, g2, nbr2, wa, wg, wf, bvec, nc, m):
    bn = _BN
    bm = bn * m
    grid = (nc // bn,)

    def body(atom_ref, g_ref, nbr_ref, wa_ref, wg_ref, wf_ref, b_ref,
             s1_ref, s2_ref):
        y = _edge_y(atom_ref, g_ref, nbr_ref, wa_ref, wg_ref, wf_ref, b_ref,
                    bn, m)
        s1 = jnp.sum(y, axis=(0, 1))[None, :]
        s2 = jnp.sum(y * y, axis=(0, 1))[None, :]

        @pl.when(pl.program_id(0) == 0)
        def _():
            s1_ref[...] = jnp.zeros_like(s1_ref)
            s2_ref[...] = jnp.zeros_like(s2_ref)

        s1_ref[...] += s1
        s2_ref[...] += s2

    return pl.pallas_call(
        body,
        grid=grid,
        in_specs=[
            pl.BlockSpec((bn, 64), lambda i: (i, 0)),
            pl.BlockSpec((bm, 64), lambda i: (i, 0)),
            pl.BlockSpec((bm, 41), lambda i: (i, 0)),
            pl.BlockSpec((64, 128), lambda i: (0, 0)),
            pl.BlockSpec((64, 128), lambda i: (0, 0)),
            pl.BlockSpec((41, 128), lambda i: (0, 0)),
            pl.BlockSpec((1, 128), lambda i: (0, 0)),
        ],
        out_specs=[
            pl.BlockSpec((1, 128), lambda i: (0, 0)),
            pl.BlockSpec((1, 128), lambda i: (0, 0)),
        ],
        out_shape=[jax.ShapeDtypeStruct((1, 128), jnp.float32)] * 2,
    )(atom, g2, nbr2, wa, wg, wf, bvec)


def _pass2(atom, g2, nbr2, wa, wg, wf, bvec, nc, m):
    bn = _BN
    bm = bn * m
    grid = (nc // bn,)

    def body(atom_ref, g_ref, nbr_ref, wa_ref, wg_ref, wf_ref, b_ref,
             s_ref, t1_ref, t2_ref):
        yb = _edge_y(atom_ref, g_ref, nbr_ref, wa_ref, wg_ref, wf_ref, b_ref,
                     bn, m)
        log2e = 1.4426950408889634
        ln2 = 0.6931471805599453
        filt = 1.0 / (1.0 + jnp.exp2(yb[..., :64] * (-log2e)))
        core = jnp.log2(1.0 + jnp.exp2(yb[..., 64:] * log2e)) * ln2
        sb = jnp.sum(filt * core, axis=1)
        s_ref[...] = sb
        t1 = jnp.sum(sb, axis=0)[None, :]
        t2 = jnp.sum(sb * sb, axis=0)[None, :]

        @pl.when(pl.program_id(0) == 0)
        def _():
            t1_ref[...] = jnp.zeros_like(t1_ref)
            t2_ref[...] = jnp.zeros_like(t2_ref)

        t1_ref[...] += t1
        t2_ref[...] += t2

    return pl.pallas_call(
        body,
        grid=grid,
        in_specs=[
            pl.BlockSpec((bn, 64), lambda i: (i, 0)),
            pl.BlockSpec((bm, 64), lambda i: (i, 0)),
            pl.BlockSpec((bm, 41), lambda i: (i, 0)),
            pl.BlockSpec((64, 128), lambda i: (0, 0)),
            pl.BlockSpec((64, 128), lambda i: (0, 0)),
            pl.BlockSpec((41, 128), lambda i: (0, 0)),
            pl.BlockSpec((1, 128), lambda i: (0, 0)),
        ],
        out_specs=[
            pl.BlockSpec((bn, 64), lambda i: (i, 0)),
            pl.BlockSpec((1, 64), lambda i: (0, 0)),
            pl.BlockSpec((1, 64), lambda i: (0, 0)),
        ],
        out_shape=[
            jax.ShapeDtypeStruct((nc, 64), jnp.float32),
            jax.ShapeDtypeStruct((1, 64), jnp.float32),
            jax.ShapeDtypeStruct((1, 64), jnp.float32),
        ],
    )(atom, g2, nbr2, wa, wg, wf, bvec)


def _softplus(x):
    return jnp.maximum(x, 0.0) + jnp.log1p(jnp.exp(-jnp.abs(x)))


def _pass3(atom, s, scale2, shift2, n):
    bn = 5000
    grid = (n // bn,)

    def body(atom_ref, s_ref, sc_ref, sh_ref, out_ref):
        z = atom_ref[...] + s_ref[...] * sc_ref[...] + sh_ref[...]
        out_ref[...] = _softplus(z)

    return pl.pallas_call(
        body,
        grid=grid,
        in_specs=[
            pl.BlockSpec((bn, 64), lambda i: (i, 0)),
            pl.BlockSpec((bn, 64), lambda i: (i, 0)),
            pl.BlockSpec((1, 64), lambda i: (0, 0)),
            pl.BlockSpec((1, 64), lambda i: (0, 0)),
        ],
        out_specs=pl.BlockSpec((bn, 64), lambda i: (i, 0)),
        out_shape=jax.ShapeDtypeStruct((n, 64), jnp.float32),
    )(atom, s, scale2, shift2)


def kernel(atom_fea, nbr_fea, nbr_fea_idx, W_full, b_full,
           bn1_gamma, bn1_beta, bn2_gamma, bn2_beta):
    n, m = nbr_fea_idx.shape
    idx_flat = nbr_fea_idx.reshape(-1).astype(jnp.int32)
    nbr2 = nbr_fea.reshape(n * m, 41).astype(jnp.bfloat16)

    wa = W_full[:, :64].T
    wg = W_full[:, 64:128].T
    wf = W_full[:, 128:].T.astype(jnp.bfloat16)
    bvec = b_full.reshape(1, 128)

    g2 = _sc_gather(atom_fea, idx_flat)

    s1, s2 = _pass1(atom_fea, g2, nbr2, wa, wg, wf, bvec, n, m)
    cnt = float(n * m)
    mean1 = s1 / cnt
    var1 = s2 / cnt - mean1 * mean1
    scale1 = bn1_gamma[None, :] * lax.rsqrt(var1 + EPS)
    shift1 = bn1_beta[None, :] - mean1 * scale1

    wa_2 = wa * scale1
    wg_2 = wg * scale1
    wf_2 = (wf.astype(jnp.float32) * scale1).astype(jnp.bfloat16)
    b_2 = bvec * scale1 + shift1

    s, t1, t2 = _pass2(atom_fea, g2, nbr2, wa_2, wg_2, wf_2, b_2, n, m)
    mean2 = t1 / float(n)
    var2 = t2 / float(n) - mean2 * mean2
    scale2 = bn2_gamma[None, :] * lax.rsqrt(var2 + EPS)
    shift2 = bn2_beta[None, :] - mean2 * scale2

    return _pass3(atom_fea, s, scale2, shift2, n)


# speedup vs baseline: 1.1090x; 1.0200x over previous
import functools

import jax
import jax.numpy as jnp
from jax import lax
from jax.experimental import pallas as pl
from jax.experimental.pallas import tpu as pltpu
from jax.experimental.pallas import tpu_sc as plsc

EPS = 1e-5

_BN = 400
_CHUNKS = 5


def _sc_gather(table, idx_flat):
    n_rows, d = table.shape
    b = idx_flat.shape[0]
    nw = 32
    bpw = b // nw
    ch = 200
    nit = bpw // ch
    mesh = plsc.VectorSubcoreMesh(core_axis_name="c", subcore_axis_name="s")

    @functools.partial(
        pl.kernel,
        mesh=mesh,
        compiler_params=pltpu.CompilerParams(use_tc_tiling_on_sc=False),
        out_type=jax.ShapeDtypeStruct((b, d), table.dtype),
        scratch_types=(
            [pltpu.VMEM((ch,), jnp.int32) for _ in range(4)] +
            [pltpu.VMEM((ch, d), table.dtype) for _ in range(4)] +
            [pltpu.SemaphoreType.DMA for _ in range(8)]
        ),
    )
    def gather_kernel(table_hbm, idx_hbm, out_hbm,
                      ix0, ix1, ix2, ix3, rv0, rv1, rv2, rv3,
                      sg0, sg1, sg2, sg3, so0, so1, so2, so3):
        wid = lax.axis_index("s") * 2 + lax.axis_index("c")
        base = wid * bpw
        ixs = (ix0, ix1, ix2, ix3)
        rvs = (rv0, rv1, rv2, rv3)
        sgs = (sg0, sg1, sg2, sg3)
        sos = (so0, so1, so2, so3)

        def body(j, carry):
            offs = [base + (4 * j + k) * ch for k in range(4)]
            pltpu.sync_copy(idx_hbm.at[pl.ds(offs[0], ch)], ixs[0])
            g_prev = pltpu.async_copy(table_hbm.at[ixs[0]], rvs[0], sgs[0])
            outs = []
            for k in range(1, 4):
                pltpu.sync_copy(idx_hbm.at[pl.ds(offs[k], ch)], ixs[k])
                g_prev.wait()
                outs.append(pltpu.async_copy(
                    rvs[k - 1], out_hbm.at[pl.ds(offs[k - 1], ch)],
                    sos[k - 1]))
                g_prev = pltpu.async_copy(table_hbm.at[ixs[k]], rvs[k], sgs[k])
            g_prev.wait()
            outs.append(pltpu.async_copy(
                rvs[3], out_hbm.at[pl.ds(offs[3], ch)], sos[3]))
            for o in outs:
                o.wait()
            return carry

        lax.fori_loop(0, nit // 4, body, 0)

        def tail(i, carry):
            off = base + i * ch
            pltpu.sync_copy(idx_hbm.at[pl.ds(off, ch)], ix0)
            pltpu.async_copy(table_hbm.at[ix0], rv0, sg0).wait()
            pltpu.sync_copy(rv0, out_hbm.at[pl.ds(off, ch)])
            return carry

        lax.fori_loop((nit // 4) * 4, nit, tail, 0)

    return gather_kernel(table, idx_flat)


def _edge_y(atom_ref, g_ref, nbr_ref, wa_ref, wg_ref, wf_ref, b_ref, bn, m):
    ya = jnp.dot(atom_ref[...], wa_ref[...], preferred_element_type=jnp.float32)
    yg = jnp.dot(g_ref[...], wg_ref[...], preferred_element_type=jnp.float32)
    yf = jnp.dot(nbr_ref[...], wf_ref[...], preferred_element_type=jnp.float32)
    y = (yg + yf + b_ref[...]).reshape(bn, m, 128)
    return y + ya[:, None, :]


def _pass1(atom, g2, nbr2, wa, wg, wf, bvec, nc, m):
    bn = _BN
    bm = bn * m
    grid = (nc // bn,)

    def body(atom_ref, g_ref, nbr_ref, wa_ref, wg_ref, wf_ref, b_ref,
             s1_ref, s2_ref):
        y = _edge_y(atom_ref, g_ref, nbr_ref, wa_ref, wg_ref, wf_ref, b_ref,
                    bn, m)
        s1 = jnp.sum(y, axis=(0, 1))[None, :]
        s2 = jnp.sum(y * y, axis=(0, 1))[None, :]

        @pl.when(pl.program_id(0) == 0)
        def _():
            s1_ref[...] = jnp.zeros_like(s1_ref)
            s2_ref[...] = jnp.zeros_like(s2_ref)

        s1_ref[...] += s1
        s2_ref[...] += s2

    return pl.pallas_call(
        body,
        grid=grid,
        in_specs=[
            pl.BlockSpec((bn, 64), lambda i: (i, 0)),
            pl.BlockSpec((bm, 64), lambda i: (i, 0)),
            pl.BlockSpec((bm, 41), lambda i: (i, 0)),
            pl.BlockSpec((64, 128), lambda i: (0, 0)),
            pl.BlockSpec((64, 128), lambda i: (0, 0)),
            pl.BlockSpec((41, 128), lambda i: (0, 0)),
            pl.BlockSpec((1, 128), lambda i: (0, 0)),
        ],
        out_specs=[
            pl.BlockSpec((1, 128), lambda i: (0, 0)),
            pl.BlockSpec((1, 128), lambda i: (0, 0)),
        ],
        out_shape=[jax.ShapeDtypeStruct((1, 128), jnp.float32)] * 2,
    )(atom, g2, nbr2, wa, wg, wf, bvec)


def _pass2(atom, g2, nbr2, wa, wg, wf, bvec, nc, m):
    bn = _BN
    bm = bn * m
    grid = (nc // bn,)

    def body(atom_ref, g_ref, nbr_ref, wa_ref, wg_ref, wf_ref, b_ref,
             s_ref, t1_ref, t2_ref):
        yb = _edge_y(atom_ref, g_ref, nbr_ref, wa_ref, wg_ref, wf_ref, b_ref,
                     bn, m)
        log2e = 1.4426950408889634
        ln2 = 0.6931471805599453
        filt = 1.0 / (1.0 + jnp.exp2(yb[..., :64] * (-log2e)))
        core = jnp.log2(1.0 + jnp.exp2(yb[..., 64:] * log2e)) * ln2
        sb = jnp.sum(filt * core, axis=1)
        s_ref[...] = sb
        t1 = jnp.sum(sb, axis=0)[None, :]
        t2 = jnp.sum(sb * sb, axis=0)[None, :]

        @pl.when(pl.program_id(0) == 0)
        def _():
            t1_ref[...] = jnp.zeros_like(t1_ref)
            t2_ref[...] = jnp.zeros_like(t2_ref)

        t1_ref[...] += t1
        t2_ref[...] += t2

    return pl.pallas_call(
        body,
        grid=grid,
        in_specs=[
            pl.BlockSpec((bn, 64), lambda i: (i, 0)),
            pl.BlockSpec((bm, 64), lambda i: (i, 0)),
            pl.BlockSpec((bm, 41), lambda i: (i, 0)),
            pl.BlockSpec((64, 128), lambda i: (0, 0)),
            pl.BlockSpec((64, 128), lambda i: (0, 0)),
            pl.BlockSpec((41, 128), lambda i: (0, 0)),
            pl.BlockSpec((1, 128), lambda i: (0, 0)),
        ],
        out_specs=[
            pl.BlockSpec((bn, 64), lambda i: (i, 0)),
            pl.BlockSpec((1, 64), lambda i: (0, 0)),
            pl.BlockSpec((1, 64), lambda i: (0, 0)),
        ],
        out_shape=[
            jax.ShapeDtypeStruct((nc, 64), jnp.float32),
            jax.ShapeDtypeStruct((1, 64), jnp.float32),
            jax.ShapeDtypeStruct((1, 64), jnp.float32),
        ],
    )(atom, g2, nbr2, wa, wg, wf, bvec)


def _softplus(x):
    return jnp.maximum(x, 0.0) + jnp.log1p(jnp.exp(-jnp.abs(x)))


def _pass3(atom, s, scale2, shift2, n):
    bn = 5000
    grid = (n // bn,)

    def body(atom_ref, s_ref, sc_ref, sh_ref, out_ref):
        z = atom_ref[...] + s_ref[...] * sc_ref[...] + sh_ref[...]
        out_ref[...] = _softplus(z)

    return pl.pallas_call(
        body,
        grid=grid,
        in_specs=[
            pl.BlockSpec((bn, 64), lambda i: (i, 0)),
            pl.BlockSpec((bn, 64), lambda i: (i, 0)),
            pl.BlockSpec((1, 64), lambda i: (0, 0)),
            pl.BlockSpec((1, 64), lambda i: (0, 0)),
        ],
        out_specs=pl.BlockSpec((bn, 64), lambda i: (i, 0)),
        out_shape=jax.ShapeDtypeStruct((n, 64), jnp.float32),
    )(atom, s, scale2, shift2)


def kernel(atom_fea, nbr_fea, nbr_fea_idx, W_full, b_full,
           bn1_gamma, bn1_beta, bn2_gamma, bn2_beta):
    n, m = nbr_fea_idx.shape
    idx_flat = nbr_fea_idx.reshape(-1).astype(jnp.int32)
    nbr2 = nbr_fea.reshape(n * m, 41).astype(jnp.bfloat16)

    wa = W_full[:, :64].T
    wg = W_full[:, 64:128].T
    wf = W_full[:, 128:].T.astype(jnp.bfloat16)
    bvec = b_full.reshape(1, 128)

    g2 = _sc_gather(atom_fea, idx_flat)

    s1, s2 = _pass1(atom_fea, g2, nbr2, wa, wg, wf, bvec, n, m)
    cnt = float(n * m)
    mean1 = s1 / cnt
    var1 = s2 / cnt - mean1 * mean1
    scale1 = bn1_gamma[None, :] * lax.rsqrt(var1 + EPS)
    shift1 = bn1_beta[None, :] - mean1 * scale1

    wa_2 = wa * scale1
    wg_2 = wg * scale1
    wf_2 = (wf.astype(jnp.float32) * scale1).astype(jnp.bfloat16)
    b_2 = bvec * scale1 + shift1

    s, t1, t2 = _pass2(atom_fea, g2, nbr2, wa_2, wg_2, wf_2, b_2, n, m)
    mean2 = t1 / float(n)
    var2 = t2 / float(n) - mean2 * mean2
    scale2 = bn2_gamma[None, :] * lax.rsqrt(var2 + EPS)
    shift2 = bn2_beta[None, :] - mean2 * scale2

    return _pass3(atom_fea, s, scale2, shift2, n)


# speedup vs baseline: 1.1731x; 1.0578x over previous
import functools

import jax
import jax.numpy as jnp
from jax import lax
from jax.experimental import pallas as pl
from jax.experimental.pallas import tpu as pltpu
from jax.experimental.pallas import tpu_sc as plsc

EPS = 1e-5

_BN = 1000
_CHUNKS = 5


def _sc_gather(table, idx_flat):
    n_rows, d = table.shape
    b = idx_flat.shape[0]
    nw = 32
    bpw = b // nw
    ch = 200
    nit = bpw // ch
    mesh = plsc.VectorSubcoreMesh(core_axis_name="c", subcore_axis_name="s")

    @functools.partial(
        pl.kernel,
        mesh=mesh,
        compiler_params=pltpu.CompilerParams(use_tc_tiling_on_sc=False),
        out_type=jax.ShapeDtypeStruct((b, d), table.dtype),
        scratch_types=(
            [pltpu.VMEM((ch,), jnp.int32) for _ in range(4)] +
            [pltpu.VMEM((ch, d), table.dtype) for _ in range(4)] +
            [pltpu.SemaphoreType.DMA for _ in range(8)]
        ),
    )
    def gather_kernel(table_hbm, idx_hbm, out_hbm,
                      ix0, ix1, ix2, ix3, rv0, rv1, rv2, rv3,
                      sg0, sg1, sg2, sg3, so0, so1, so2, so3):
        wid = lax.axis_index("s") * 2 + lax.axis_index("c")
        base = wid * bpw
        ixs = (ix0, ix1, ix2, ix3)
        rvs = (rv0, rv1, rv2, rv3)
        sgs = (sg0, sg1, sg2, sg3)
        sos = (so0, so1, so2, so3)

        def body(j, carry):
            offs = [base + (4 * j + k) * ch for k in range(4)]
            pltpu.sync_copy(idx_hbm.at[pl.ds(offs[0], ch)], ixs[0])
            g_prev = pltpu.async_copy(table_hbm.at[ixs[0]], rvs[0], sgs[0])
            outs = []
            for k in range(1, 4):
                pltpu.sync_copy(idx_hbm.at[pl.ds(offs[k], ch)], ixs[k])
                g_prev.wait()
                outs.append(pltpu.async_copy(
                    rvs[k - 1], out_hbm.at[pl.ds(offs[k - 1], ch)],
                    sos[k - 1]))
                g_prev = pltpu.async_copy(table_hbm.at[ixs[k]], rvs[k], sgs[k])
            g_prev.wait()
            outs.append(pltpu.async_copy(
                rvs[3], out_hbm.at[pl.ds(offs[3], ch)], sos[3]))
            for o in outs:
                o.wait()
            return carry

        lax.fori_loop(0, nit // 4, body, 0)

        def tail(i, carry):
            off = base + i * ch
            pltpu.sync_copy(idx_hbm.at[pl.ds(off, ch)], ix0)
            pltpu.async_copy(table_hbm.at[ix0], rv0, sg0).wait()
            pltpu.sync_copy(rv0, out_hbm.at[pl.ds(off, ch)])
            return carry

        lax.fori_loop((nit // 4) * 4, nit, tail, 0)

    return gather_kernel(table, idx_flat)


def _edge_y(atom_ref, g_ref, nbr_ref, wa_ref, wg_ref, wf_ref, b_ref, bn, m):
    ya = jnp.dot(atom_ref[...], wa_ref[...], preferred_element_type=jnp.float32)
    yg = jnp.dot(g_ref[...], wg_ref[...], preferred_element_type=jnp.float32)
    yf = jnp.dot(nbr_ref[...], wf_ref[...], preferred_element_type=jnp.float32)
    y = (yg + yf + b_ref[...]).reshape(bn, m, 128)
    return y + ya[:, None, :]


def _pass1(atom, g2, nbr2, wa, wg, wf, bvec, nc, m):
    bn = _BN
    bm = bn * m
    grid = (nc // bn,)

    def body(atom_ref, g_ref, nbr_ref, wa_ref, wg_ref, wf_ref, b_ref,
             s1_ref, s2_ref):
        y = _edge_y(atom_ref, g_ref, nbr_ref, wa_ref, wg_ref, wf_ref, b_ref,
                    bn, m)
        s1 = jnp.sum(y, axis=(0, 1))[None, :]
        s2 = jnp.sum(y * y, axis=(0, 1))[None, :]

        @pl.when(pl.program_id(0) == 0)
        def _():
            s1_ref[...] = jnp.zeros_like(s1_ref)
            s2_ref[...] = jnp.zeros_like(s2_ref)

        s1_ref[...] += s1
        s2_ref[...] += s2

    return pl.pallas_call(
        body,
        grid=grid,
        in_specs=[
            pl.BlockSpec((bn, 64), lambda i: (i, 0)),
            pl.BlockSpec((bm, 64), lambda i: (i, 0)),
            pl.BlockSpec((bm, 41), lambda i: (i, 0)),
            pl.BlockSpec((64, 128), lambda i: (0, 0)),
            pl.BlockSpec((64, 128), lambda i: (0, 0)),
            pl.BlockSpec((41, 128), lambda i: (0, 0)),
            pl.BlockSpec((1, 128), lambda i: (0, 0)),
        ],
        out_specs=[
            pl.BlockSpec((1, 128), lambda i: (0, 0)),
            pl.BlockSpec((1, 128), lambda i: (0, 0)),
        ],
        out_shape=[jax.ShapeDtypeStruct((1, 128), jnp.float32)] * 2,
    )(atom, g2, nbr2, wa, wg, wf, bvec)


def _pass2(atom, g2, nbr2, wa, wg, wf, bvec, nc, m):
    bn = _BN
    bm = bn * m
    grid = (nc // bn,)

    def body(atom_ref, g_ref, nbr_ref, wa_ref, wg_ref, wf_ref, b_ref,
             s_ref, t1_ref, t2_ref):
        yb = _edge_y(atom_ref, g_ref, nbr_ref, wa_ref, wg_ref, wf_ref, b_ref,
                     bn, m)
        log2e = 1.4426950408889634
        ln2 = 0.6931471805599453
        filt = 1.0 / (1.0 + jnp.exp2(yb[..., :64] * (-log2e)))
        core = jnp.log2(1.0 + jnp.exp2(yb[..., 64:] * log2e)) * ln2
        sb = jnp.sum(filt * core, axis=1)
        s_ref[...] = sb
        t1 = jnp.sum(sb, axis=0)[None, :]
        t2 = jnp.sum(sb * sb, axis=0)[None, :]

        @pl.when(pl.program_id(0) == 0)
        def _():
            t1_ref[...] = jnp.zeros_like(t1_ref)
            t2_ref[...] = jnp.zeros_like(t2_ref)

        t1_ref[...] += t1
        t2_ref[...] += t2

    return pl.pallas_call(
        body,
        grid=grid,
        in_specs=[
            pl.BlockSpec((bn, 64), lambda i: (i, 0)),
            pl.BlockSpec((bm, 64), lambda i: (i, 0)),
            pl.BlockSpec((bm, 41), lambda i: (i, 0)),
            pl.BlockSpec((64, 128), lambda i: (0, 0)),
            pl.BlockSpec((64, 128), lambda i: (0, 0)),
            pl.BlockSpec((41, 128), lambda i: (0, 0)),
            pl.BlockSpec((1, 128), lambda i: (0, 0)),
        ],
        out_specs=[
            pl.BlockSpec((bn, 64), lambda i: (i, 0)),
            pl.BlockSpec((1, 64), lambda i: (0, 0)),
            pl.BlockSpec((1, 64), lambda i: (0, 0)),
        ],
        out_shape=[
            jax.ShapeDtypeStruct((nc, 64), jnp.float32),
            jax.ShapeDtypeStruct((1, 64), jnp.float32),
            jax.ShapeDtypeStruct((1, 64), jnp.float32),
        ],
    )(atom, g2, nbr2, wa, wg, wf, bvec)


def _softplus(x):
    return jnp.maximum(x, 0.0) + jnp.log1p(jnp.exp(-jnp.abs(x)))


def _pass3(atom, s, scale2, shift2, n):
    bn = 5000
    grid = (n // bn,)

    def body(atom_ref, s_ref, sc_ref, sh_ref, out_ref):
        z = atom_ref[...] + s_ref[...] * sc_ref[...] + sh_ref[...]
        out_ref[...] = _softplus(z)

    return pl.pallas_call(
        body,
        grid=grid,
        in_specs=[
            pl.BlockSpec((bn, 64), lambda i: (i, 0)),
            pl.BlockSpec((bn, 64), lambda i: (i, 0)),
            pl.BlockSpec((1, 64), lambda i: (0, 0)),
            pl.BlockSpec((1, 64), lambda i: (0, 0)),
        ],
        out_specs=pl.BlockSpec((bn, 64), lambda i: (i, 0)),
        out_shape=jax.ShapeDtypeStruct((n, 64), jnp.float32),
    )(atom, s, scale2, shift2)


def kernel(atom_fea, nbr_fea, nbr_fea_idx, W_full, b_full,
           bn1_gamma, bn1_beta, bn2_gamma, bn2_beta):
    n, m = nbr_fea_idx.shape
    idx_flat = nbr_fea_idx.reshape(-1).astype(jnp.int32)
    nbr2 = nbr_fea.reshape(n * m, 41).astype(jnp.bfloat16)

    wa = W_full[:, :64].T
    wg = W_full[:, 64:128].T
    wf = W_full[:, 128:].T.astype(jnp.bfloat16)
    bvec = b_full.reshape(1, 128)

    g2 = _sc_gather(atom_fea, idx_flat)

    s1, s2 = _pass1(atom_fea, g2, nbr2, wa, wg, wf, bvec, n, m)
    cnt = float(n * m)
    mean1 = s1 / cnt
    var1 = s2 / cnt - mean1 * mean1
    scale1 = bn1_gamma[None, :] * lax.rsqrt(var1 + EPS)
    shift1 = bn1_beta[None, :] - mean1 * scale1

    wa_2 = wa * scale1
    wg_2 = wg * scale1
    wf_2 = (wf.astype(jnp.float32) * scale1).astype(jnp.bfloat16)
    b_2 = bvec * scale1 + shift1

    s, t1, t2 = _pass2(atom_fea, g2, nbr2, wa_2, wg_2, wf_2, b_2, n, m)
    mean2 = t1 / float(n)
    var2 = t2 / float(n) - mean2 * mean2
    scale2 = bn2_gamma[None, :] * lax.rsqrt(var2 + EPS)
    shift2 = bn2_beta[None, :] - mean2 * scale2

    return _pass3(atom_fea, s, scale2, shift2, n)
